# baseline (speedup 1.0000x reference)
import jax
import jax.numpy as jnp
from jax.experimental import pallas as pl
from jax.experimental.pallas import tpu as pltpu

E = 8
FFN = 4096
H = 1024
T = 512
BF = 1024
NBF = FFN // BF


def _glu_body(eidx_ref, x_ref, w1_ref, v1_ref, w2_ref, o_ref, g_ref):
    f = pl.program_id(0)
    x = x_ref[...]
    c = jax.lax.dot_general(
        x, w1_ref[...], (((1,), (1,)), ((), ())),
        preferred_element_type=jnp.float32)
    c = c + jax.lax.dot_general(
        x, v1_ref[...], (((1,), (1,)), ((), ())),
        preferred_element_type=jnp.float32)
    c = c + jax.lax.dot_general(
        x, w2_ref[...], (((1,), (1,)), ((), ())),
        preferred_element_type=jnp.float32)

    @pl.when(f == 0)
    def _():
        o_ref[...] = c

    @pl.when(f != 0)
    def _():
        o_ref[...] = o_ref[...] + c


def kernel(x, expert_idx, w1, v1, w2):
    eidx = jnp.asarray(expert_idx, dtype=jnp.int32).reshape((1,))

    def _up_map(f, e):
        return (e[0] * NBF + jnp.minimum(f, NBF - 1), 0)

    def _down_map(f, e):
        return (e[0] * NBF + jnp.maximum(f - 1, 0), 0)

    grid_spec = pltpu.PrefetchScalarGridSpec(
        num_scalar_prefetch=1,
        grid=(NBF + 1,),
        in_specs=[
            pl.BlockSpec((T, H), lambda f, e: (0, 0)),
            pl.BlockSpec((BF, H), _up_map),
            pl.BlockSpec((BF, H), _up_map),
            pl.BlockSpec((BF, H), _down_map),
        ],
        out_specs=pl.BlockSpec((T, H), lambda f, e: (0, 0)),
        scratch_shapes=[pltpu.VMEM((T, BF), jnp.float32)],
    )

    return pl.pallas_call(
        _glu_body,
        grid_spec=grid_spec,
        out_shape=jax.ShapeDtypeStruct((T, H), jnp.float32),
        compiler_params=pltpu.CompilerParams(
            dimension_semantics=("arbitrary",)),
    )(eidx, x, w1, v1, w2)


# speedup vs baseline: 1.1322x; 1.1322x over previous
import jax
import jax.numpy as jnp
from jax.experimental import pallas as pl
from jax.experimental.pallas import tpu as pltpu

E = 8
FFN = 4096
H = 1024
T = 512
BF = 1024
NBF = FFN // BF


def _glu_body(eidx_ref, x_ref, w1_ref, v1_ref, w2_ref, o_ref, g_ref):
    f = pl.program_id(0)

    @pl.when(f == 1)
    def _():
        o_ref[...] = jax.lax.dot_general(
            g_ref[...], w2_ref[...].astype(jnp.bfloat16),
            (((1,), (0,)), ((), ())),
            preferred_element_type=jnp.float32)

    @pl.when(f > 1)
    def _():
        o_ref[...] = o_ref[...] + jax.lax.dot_general(
            g_ref[...], w2_ref[...].astype(jnp.bfloat16),
            (((1,), (0,)), ((), ())),
            preferred_element_type=jnp.float32)

    @pl.when(f < NBF)
    def _():
        x = x_ref[...]
        h1 = jax.lax.dot_general(
            x, w1_ref[...], (((1,), (1,)), ((), ())),
            preferred_element_type=jnp.float32)
        h2 = jax.lax.dot_general(
            x, v1_ref[...], (((1,), (1,)), ((), ())),
            preferred_element_type=jnp.float32)
        g_ref[...] = (h1 * jax.lax.logistic(h1) * h2).astype(jnp.bfloat16)


def kernel(x, expert_idx, w1, v1, w2):
    eidx = jnp.asarray(expert_idx, dtype=jnp.int32).reshape((1,))

    def _up_map(f, e):
        return (e[0] * NBF + jnp.minimum(f, NBF - 1), 0)

    def _down_map(f, e):
        return (e[0] * NBF + jnp.maximum(f - 1, 0), 0)

    grid_spec = pltpu.PrefetchScalarGridSpec(
        num_scalar_prefetch=1,
        grid=(NBF + 1,),
        in_specs=[
            pl.BlockSpec((T, H), lambda f, e: (0, 0)),
            pl.BlockSpec((BF, H), _up_map),
            pl.BlockSpec((BF, H), _up_map),
            pl.BlockSpec((BF, H), _down_map),
        ],
        out_specs=pl.BlockSpec((T, H), lambda f, e: (0, 0)),
        scratch_shapes=[pltpu.VMEM((T, BF), jnp.bfloat16)],
    )

    return pl.pallas_call(
        _glu_body,
        grid_spec=grid_spec,
        out_shape=jax.ShapeDtypeStruct((T, H), jnp.float32),
        compiler_params=pltpu.CompilerParams(
            dimension_semantics=("arbitrary",)),
    )(eidx, x, w1, v1, w2)


# speedup vs baseline: 1.1358x; 1.0032x over previous
import jax
import jax.numpy as jnp
from jax.experimental import pallas as pl
from jax.experimental.pallas import tpu as pltpu

E = 8
FFN = 4096
H = 1024
T = 512
BF = 1024
NBF = FFN // BF


def _glu_body(eidx_ref, x_ref, w1_ref, v1_ref, w2_ref, o_ref, g_ref):
    f = pl.program_id(0)

    @pl.when(f == 1)
    def _():
        o_ref[...] = jax.lax.dot_general(
            g_ref[...], w2_ref[...], (((1,), (0,)), ((), ())),
            preferred_element_type=jnp.float32)

    @pl.when(f > 1)
    def _():
        o_ref[...] = o_ref[...] + jax.lax.dot_general(
            g_ref[...], w2_ref[...], (((1,), (0,)), ((), ())),
            preferred_element_type=jnp.float32)

    @pl.when(f < NBF)
    def _():
        x = x_ref[...]
        h1 = jax.lax.dot_general(
            x, w1_ref[...], (((1,), (1,)), ((), ())),
            preferred_element_type=jnp.float32)
        h2 = jax.lax.dot_general(
            x, v1_ref[...], (((1,), (1,)), ((), ())),
            preferred_element_type=jnp.float32)
        g_ref[...] = h1 * jax.lax.logistic(h1) * h2


def kernel(x, expert_idx, w1, v1, w2):
    eidx = jnp.asarray(expert_idx, dtype=jnp.int32).reshape((1,))

    def _up_map(f, e):
        return (e[0] * NBF + jnp.minimum(f, NBF - 1), 0)

    def _down_map(f, e):
        return (e[0] * NBF + jnp.maximum(f - 1, 0), 0)

    grid_spec = pltpu.PrefetchScalarGridSpec(
        num_scalar_prefetch=1,
        grid=(NBF + 1,),
        in_specs=[
            pl.BlockSpec((T, H), lambda f, e: (0, 0)),
            pl.BlockSpec((BF, H), _up_map),
            pl.BlockSpec((BF, H), _up_map),
            pl.BlockSpec((BF, H), _down_map),
        ],
        out_specs=pl.BlockSpec((T, H), lambda f, e: (0, 0)),
        scratch_shapes=[pltpu.VMEM((T, BF), jnp.float32)],
    )

    return pl.pallas_call(
        _glu_body,
        grid_spec=grid_spec,
        out_shape=jax.ShapeDtypeStruct((T, H), jnp.float32),
        compiler_params=pltpu.CompilerParams(
            dimension_semantics=("arbitrary",)),
    )(eidx, x, w1, v1, w2)
